# baseline (speedup 1.0000x reference)
import functools

import jax
import jax.numpy as jnp
from jax import lax
from jax.experimental import pallas as pl
from jax.experimental.pallas import tpu as pltpu
from jax.experimental.pallas import tpu_sc as plsc

NUM_SEG = 512
N_ROWS = 320000
D = 128
NC = 2
NS = 16
NW = NC * NS
ROWS_PER_W = N_ROWS // NW
SUB = 80
NSUB = ROWS_PER_W // SUB
NPAIR = (NSUB - 1) // 2
SEG_PER_TILE = NUM_SEG // NS
SEG_PER_SEARCH = NUM_SEG // NW
SEARCH_STEPS = 19


def _sc_body(x_hbm, b_hbm, z_hbm, outs_hbm, outc_hbm,
             bufA, bufB, idxA, idxB, zbuf_v, obuf_v, cbuf_v, mid_v, val_v,
             semA, semB, semSA, semSB, semG, acc_sh):
    c = lax.axis_index("c")
    s = lax.axis_index("s")
    wid = s * NC + c
    seg0 = s * SEG_PER_TILE
    row0 = wid * ROWS_PER_W

    def read(j, buf, idx, sem):
        base = row0 + j * SUB
        pltpu.async_copy(x_hbm.at[pl.ds(base, SUB)], buf, sem)
        pltpu.async_copy(b_hbm.at[pl.ds(base, SUB)], idx, sem)

    def wait(buf, idx, sem):
        pltpu.make_async_copy(x_hbm.at[pl.ds(0, SUB)], buf, sem).wait()
        pltpu.make_async_copy(b_hbm.at[pl.ds(0, SUB)], idx, sem).wait()

    def scatter(buf, idx):
        pltpu.sync_copy(buf, acc_sh.at[idx], add=True)

    read(0, bufA, idxA, semA)
    read(1, bufB, idxB, semB)

    pltpu.sync_copy(z_hbm, zbuf_v)
    pltpu.sync_copy(zbuf_v, acc_sh.at[pl.ds(seg0, SEG_PER_TILE)])
    plsc.subcore_barrier()

    def body(g, carry):
        wait(bufA, idxA, semA)
        scat_a = pltpu.async_copy(bufA, acc_sh.at[idxA], semSA, add=True)
        wait(bufB, idxB, semB)
        scat_b = pltpu.async_copy(bufB, acc_sh.at[idxB], semSB, add=True)
        scat_a.wait()

        @pl.when(g < NPAIR - 1)
        def _():
            read(2 * g + 2, bufA, idxA, semA)

        scat_b.wait()

        @pl.when(g < NPAIR - 1)
        def _():
            read(2 * g + 3, bufB, idxB, semB)

        return carry

    lax.fori_loop(0, NPAIR, body, 0)
    read(NSUB - 1, bufA, idxA, semA)
    wait(bufA, idxA, semA)
    scatter(bufA, idxA)

    cseg0 = (NUM_SEG // NC) * c + SEG_PER_SEARCH * s
    segv = cseg0 + lax.iota(jnp.int32, 16)
    tgt0 = segv
    tgt1 = segv + 1
    zero = jnp.zeros((16,), jnp.int32)
    nfull = zero + N_ROWS

    def step(k, st):
        lo0, hi0, lo1, hi1 = st
        mid0 = jnp.minimum(lax.shift_right_logical(lo0 + hi0, 1), N_ROWS - 1)
        mid1 = jnp.minimum(lax.shift_right_logical(lo1 + hi1, 1), N_ROWS - 1)
        mid_v[pl.ds(0, 16)] = mid0
        mid_v[pl.ds(16, 16)] = mid1
        pltpu.async_copy(b_hbm.at[mid_v], val_v, semG).wait()
        v0 = val_v[pl.ds(0, 16)]
        v1 = val_v[pl.ds(16, 16)]
        p0 = v0 < tgt0
        p1 = v1 < tgt1
        lo0n = jnp.where(p0, mid0 + 1, lo0)
        hi0n = jnp.where(p0, hi0, mid0)
        lo1n = jnp.where(p1, mid1 + 1, lo1)
        hi1n = jnp.where(p1, hi1, mid1)
        return (lo0n, hi0n, lo1n, hi1n)

    lo0, _, lo1, _ = lax.fori_loop(
        0, SEARCH_STEPS, step, (zero, nfull, zero, nfull))
    cbuf_v[pl.ds(0, 16)] = (lo1 - lo0).astype(jnp.float32)
    pltpu.sync_copy(cbuf_v, outc_hbm.at[pl.ds(cseg0, SEG_PER_SEARCH)])

    plsc.subcore_barrier()
    out0 = c * NUM_SEG + seg0
    pltpu.sync_copy(acc_sh.at[pl.ds(seg0, SEG_PER_TILE)], obuf_v)
    pltpu.sync_copy(obuf_v, outs_hbm.at[pl.ds(out0, SEG_PER_TILE)])


@functools.partial(
    pl.kernel,
    out_type=(
        jax.ShapeDtypeStruct((NC * NUM_SEG, D), jnp.float32),
        jax.ShapeDtypeStruct((NUM_SEG,), jnp.float32),
    ),
    mesh=plsc.VectorSubcoreMesh(core_axis_name="c", subcore_axis_name="s"),
    scratch_types=[
        pltpu.VMEM((SUB, D), jnp.float32),
        pltpu.VMEM((SUB, D), jnp.float32),
        pltpu.VMEM((SUB,), jnp.int32),
        pltpu.VMEM((SUB,), jnp.int32),
        pltpu.VMEM((SEG_PER_TILE, D), jnp.float32),
        pltpu.VMEM((SEG_PER_TILE, D), jnp.float32),
        pltpu.VMEM((SEG_PER_SEARCH,), jnp.float32),
        pltpu.VMEM((32,), jnp.int32),
        pltpu.VMEM((32,), jnp.int32),
        pltpu.SemaphoreType.DMA,
        pltpu.SemaphoreType.DMA,
        pltpu.SemaphoreType.DMA,
        pltpu.SemaphoreType.DMA,
        pltpu.SemaphoreType.DMA,
        pltpu.VMEM_SHARED((NUM_SEG, D), jnp.float32),
    ],
)
def _sc_accumulate(x_hbm, b_hbm, z_hbm, outs_hbm, outc_hbm,
                   bufA, bufB, idxA, idxB, zbuf_v, obuf_v, cbuf_v, mid_v,
                   val_v, semA, semB, semSA, semSB, semG, acc_sh):
    _sc_body(x_hbm, b_hbm, z_hbm, outs_hbm, outc_hbm,
             bufA, bufB, idxA, idxB, zbuf_v, obuf_v, cbuf_v, mid_v, val_v,
             semA, semB, semSA, semSB, semG, acc_sh)


def _fin_body(s_ref, c_ref, o_ref):
    sums = s_ref[0] + s_ref[1]
    o_ref[...] = sums / jnp.maximum(c_ref[...], 1.0)


def kernel(x, batch):
    batch = batch.astype(jnp.int32)
    zeros = jnp.zeros((SEG_PER_TILE, D), jnp.float32)
    psums, cnts = _sc_accumulate(x, batch, zeros)
    psums = psums.reshape(NC, NUM_SEG, D)
    cnts = cnts.reshape(NUM_SEG, 1)
    return pl.pallas_call(
        _fin_body,
        out_shape=jax.ShapeDtypeStruct((NUM_SEG, D), jnp.float32),
    )(psums, cnts)


# speedup vs baseline: 1.1565x; 1.1565x over previous
import functools

import jax
import jax.numpy as jnp
from jax import lax
from jax.experimental import pallas as pl
from jax.experimental.pallas import tpu as pltpu
from jax.experimental.pallas import tpu_sc as plsc

NUM_SEG = 512
N_ROWS = 320000
D = 128
NC = 2
NS = 16
NW = NC * NS
ROWS_PER_W = N_ROWS // NW
SUB = 128
NFULL = ROWS_PER_W // SUB
TAIL = ROWS_PER_W - NFULL * SUB
NPAIR = NFULL // 2
SEG_PER_TILE = NUM_SEG // NS
SEG_PER_SEARCH = NUM_SEG // NW
SEARCH_STEPS = 19


def _sc_body(x_hbm, b_hbm, z_hbm, outs_hbm, outc_hbm,
             bufA, bufB, bufT, idxA, idxB, idxT, zbuf_v, obuf_v, cbuf_v,
             mid_v, val_v, semA, semB, semG, acc_sh):
    c = lax.axis_index("c")
    s = lax.axis_index("s")
    wid = s * NC + c
    seg0 = s * SEG_PER_TILE
    row0 = wid * ROWS_PER_W

    def read(j, buf, idx, sem):
        base = row0 + j * SUB
        pltpu.async_copy(x_hbm.at[pl.ds(base, SUB)], buf, sem)
        pltpu.async_copy(b_hbm.at[pl.ds(base, SUB)], idx, sem)

    def wait(buf, idx, sem):
        pltpu.make_async_copy(x_hbm.at[pl.ds(0, SUB)], buf, sem).wait()
        pltpu.make_async_copy(b_hbm.at[pl.ds(0, SUB)], idx, sem).wait()

    def scatter(buf, idx):
        pltpu.sync_copy(buf, acc_sh.at[idx], add=True)

    read(0, bufA, idxA, semA)
    read(1, bufB, idxB, semB)

    pltpu.sync_copy(z_hbm, zbuf_v)
    pltpu.sync_copy(zbuf_v, acc_sh.at[pl.ds(seg0, SEG_PER_TILE)])
    plsc.subcore_barrier()

    def body(g, carry):
        wait(bufA, idxA, semA)
        scatter(bufA, idxA)

        @pl.when(g < NPAIR - 1)
        def _():
            read(2 * g + 2, bufA, idxA, semA)

        wait(bufB, idxB, semB)
        scatter(bufB, idxB)

        @pl.when(g < NPAIR - 1)
        def _():
            read(2 * g + 3, bufB, idxB, semB)

        return carry

    lax.fori_loop(0, NPAIR, body, 0)
    tbase = row0 + NFULL * SUB
    pltpu.async_copy(x_hbm.at[pl.ds(tbase, TAIL)], bufT, semA)
    pltpu.async_copy(b_hbm.at[pl.ds(tbase, TAIL)], idxT, semA)
    pltpu.make_async_copy(x_hbm.at[pl.ds(0, TAIL)], bufT, semA).wait()
    pltpu.make_async_copy(b_hbm.at[pl.ds(0, TAIL)], idxT, semA).wait()
    scatter(bufT, idxT)

    cseg0 = (NUM_SEG // NC) * c + SEG_PER_SEARCH * s
    segv = cseg0 + lax.iota(jnp.int32, 16)
    tgt0 = segv
    tgt1 = segv + 1
    zero = jnp.zeros((16,), jnp.int32)
    nfull = zero + N_ROWS

    def step(k, st):
        lo0, hi0, lo1, hi1 = st
        mid0 = jnp.minimum(lax.shift_right_logical(lo0 + hi0, 1), N_ROWS - 1)
        mid1 = jnp.minimum(lax.shift_right_logical(lo1 + hi1, 1), N_ROWS - 1)
        mid_v[pl.ds(0, 16)] = mid0
        mid_v[pl.ds(16, 16)] = mid1
        pltpu.async_copy(b_hbm.at[mid_v], val_v, semG).wait()
        v0 = val_v[pl.ds(0, 16)]
        v1 = val_v[pl.ds(16, 16)]
        p0 = v0 < tgt0
        p1 = v1 < tgt1
        lo0n = jnp.where(p0, mid0 + 1, lo0)
        hi0n = jnp.where(p0, hi0, mid0)
        lo1n = jnp.where(p1, mid1 + 1, lo1)
        hi1n = jnp.where(p1, hi1, mid1)
        return (lo0n, hi0n, lo1n, hi1n)

    lo0, _, lo1, _ = lax.fori_loop(
        0, SEARCH_STEPS, step, (zero, nfull, zero, nfull))
    cbuf_v[pl.ds(0, 16)] = (lo1 - lo0).astype(jnp.float32)
    pltpu.sync_copy(cbuf_v, outc_hbm.at[pl.ds(cseg0, SEG_PER_SEARCH)])

    plsc.subcore_barrier()
    out0 = c * NUM_SEG + seg0
    pltpu.sync_copy(acc_sh.at[pl.ds(seg0, SEG_PER_TILE)], obuf_v)
    pltpu.sync_copy(obuf_v, outs_hbm.at[pl.ds(out0, SEG_PER_TILE)])


@functools.partial(
    pl.kernel,
    out_type=(
        jax.ShapeDtypeStruct((NC * NUM_SEG, D), jnp.float32),
        jax.ShapeDtypeStruct((NUM_SEG,), jnp.float32),
    ),
    mesh=plsc.VectorSubcoreMesh(core_axis_name="c", subcore_axis_name="s"),
    scratch_types=[
        pltpu.VMEM((SUB, D), jnp.float32),
        pltpu.VMEM((SUB, D), jnp.float32),
        pltpu.VMEM((TAIL, D), jnp.float32),
        pltpu.VMEM((SUB,), jnp.int32),
        pltpu.VMEM((SUB,), jnp.int32),
        pltpu.VMEM((TAIL,), jnp.int32),
        pltpu.VMEM((SEG_PER_TILE, D), jnp.float32),
        pltpu.VMEM((SEG_PER_TILE, D), jnp.float32),
        pltpu.VMEM((SEG_PER_SEARCH,), jnp.float32),
        pltpu.VMEM((32,), jnp.int32),
        pltpu.VMEM((32,), jnp.int32),
        pltpu.SemaphoreType.DMA,
        pltpu.SemaphoreType.DMA,
        pltpu.SemaphoreType.DMA,
        pltpu.VMEM_SHARED((NUM_SEG, D), jnp.float32),
    ],
)
def _sc_accumulate(x_hbm, b_hbm, z_hbm, outs_hbm, outc_hbm,
                   bufA, bufB, bufT, idxA, idxB, idxT, zbuf_v, obuf_v,
                   cbuf_v, mid_v, val_v, semA, semB, semG, acc_sh):
    _sc_body(x_hbm, b_hbm, z_hbm, outs_hbm, outc_hbm,
             bufA, bufB, bufT, idxA, idxB, idxT, zbuf_v, obuf_v, cbuf_v,
             mid_v, val_v, semA, semB, semG, acc_sh)


def _fin_body(s_ref, c_ref, o_ref):
    sums = s_ref[0] + s_ref[1]
    o_ref[...] = sums / jnp.maximum(c_ref[...], 1.0)


def kernel(x, batch):
    batch = batch.astype(jnp.int32)
    zeros = jnp.zeros((SEG_PER_TILE, D), jnp.float32)
    psums, cnts = _sc_accumulate(x, batch, zeros)
    psums = psums.reshape(NC, NUM_SEG, D)
    cnts = cnts.reshape(NUM_SEG, 1)
    return pl.pallas_call(
        _fin_body,
        out_shape=jax.ShapeDtypeStruct((NUM_SEG, D), jnp.float32),
    )(psums, cnts)


# speedup vs baseline: 1.2658x; 1.0945x over previous
import functools

import jax
import jax.numpy as jnp
from jax import lax
from jax.experimental import pallas as pl
from jax.experimental.pallas import tpu as pltpu
from jax.experimental.pallas import tpu_sc as plsc

NUM_SEG = 512
N_ROWS = 320000
D = 128
NC = 2
NS = 16
NW = NC * NS
ROWS_PER_W = N_ROWS // NW
SUB = 128
NFULL = ROWS_PER_W // SUB
TAIL = ROWS_PER_W - NFULL * SUB
NPAIR = NFULL // 2
SEG_PER_TILE = NUM_SEG // NS
SEG_PER_SEARCH = NUM_SEG // NW
SEARCH_STEPS = 19


def _sc_body(x_hbm, b_hbm, z_hbm, outs_hbm, outc_hbm,
             bufA, bufB, bufT, idxA, idxB, idxT, zbuf_v, obuf_v, cbuf_v,
             mid_v, val_v, semA, semB, semG, acc_sh):
    c = lax.axis_index("c")
    s = lax.axis_index("s")
    wid = s * NC + c
    seg0 = s * SEG_PER_TILE
    row0 = wid * ROWS_PER_W

    def read(j, buf, idx, sem):
        base = row0 + j * SUB
        pltpu.async_copy(x_hbm.at[pl.ds(base, SUB)], buf, sem)
        pltpu.async_copy(b_hbm.at[pl.ds(base, SUB)], idx, sem)

    def wait(buf, idx, sem):
        pltpu.make_async_copy(x_hbm.at[pl.ds(0, SUB)], buf, sem).wait()
        pltpu.make_async_copy(b_hbm.at[pl.ds(0, SUB)], idx, sem).wait()

    def scatter(buf, idx):
        pltpu.sync_copy(buf, acc_sh.at[idx], add=True)

    read(0, bufA, idxA, semA)
    read(1, bufB, idxB, semB)

    pltpu.sync_copy(z_hbm, zbuf_v)
    pltpu.sync_copy(zbuf_v, acc_sh.at[pl.ds(seg0, SEG_PER_TILE)])
    plsc.subcore_barrier()

    cseg0 = (NUM_SEG // NC) * c + SEG_PER_SEARCH * s
    segv = cseg0 + lax.iota(jnp.int32, 16)
    tgt0 = segv
    tgt1 = segv + 1
    zero = jnp.zeros((16,), jnp.int32)
    nfull = zero + N_ROWS

    def mids(st):
        lo0, hi0, lo1, hi1 = st
        mid0 = jnp.minimum(lax.shift_right_logical(lo0 + hi0, 1), N_ROWS - 1)
        mid1 = jnp.minimum(lax.shift_right_logical(lo1 + hi1, 1), N_ROWS - 1)
        return mid0, mid1

    def issue_gather(st):
        mid0, mid1 = mids(st)
        mid_v[pl.ds(0, 16)] = mid0
        mid_v[pl.ds(16, 16)] = mid1
        pltpu.async_copy(b_hbm.at[mid_v], val_v, semG)

    def apply_step(st):
        lo0, hi0, lo1, hi1 = st
        mid0, mid1 = mids(st)
        pltpu.make_async_copy(b_hbm.at[mid_v], val_v, semG).wait()
        v0 = val_v[pl.ds(0, 16)]
        v1 = val_v[pl.ds(16, 16)]
        p0 = v0 < tgt0
        p1 = v1 < tgt1
        lo0n = jnp.where(p0, mid0 + 1, lo0)
        hi0n = jnp.where(p0, hi0, mid0)
        lo1n = jnp.where(p1, mid1 + 1, lo1)
        hi1n = jnp.where(p1, hi1, mid1)
        return (lo0n, hi0n, lo1n, hi1n)

    st0 = (zero, nfull, zero, nfull)
    issue_gather(st0)

    def body(g, st):
        st = apply_step(st)
        issue_gather(st)
        wait(bufA, idxA, semA)
        scatter(bufA, idxA)

        @pl.when(g < NPAIR - 1)
        def _():
            read(2 * g + 2, bufA, idxA, semA)

        wait(bufB, idxB, semB)
        scatter(bufB, idxB)

        @pl.when(g < NPAIR - 1)
        def _():
            read(2 * g + 3, bufB, idxB, semB)

        return st

    st = lax.fori_loop(0, NPAIR, body, st0)
    tbase = row0 + NFULL * SUB
    pltpu.async_copy(x_hbm.at[pl.ds(tbase, TAIL)], bufT, semA)
    pltpu.async_copy(b_hbm.at[pl.ds(tbase, TAIL)], idxT, semA)
    pltpu.make_async_copy(x_hbm.at[pl.ds(0, TAIL)], bufT, semA).wait()
    pltpu.make_async_copy(b_hbm.at[pl.ds(0, TAIL)], idxT, semA).wait()
    scatter(bufT, idxT)

    lo0, _, lo1, _ = apply_step(st)
    cbuf_v[pl.ds(0, 16)] = (lo1 - lo0).astype(jnp.float32)
    pltpu.sync_copy(cbuf_v, outc_hbm.at[pl.ds(cseg0, SEG_PER_SEARCH)])

    plsc.subcore_barrier()
    out0 = c * NUM_SEG + seg0
    pltpu.sync_copy(acc_sh.at[pl.ds(seg0, SEG_PER_TILE)], obuf_v)
    pltpu.sync_copy(obuf_v, outs_hbm.at[pl.ds(out0, SEG_PER_TILE)])


@functools.partial(
    pl.kernel,
    out_type=(
        jax.ShapeDtypeStruct((NC * NUM_SEG, D), jnp.float32),
        jax.ShapeDtypeStruct((NUM_SEG,), jnp.float32),
    ),
    mesh=plsc.VectorSubcoreMesh(core_axis_name="c", subcore_axis_name="s"),
    scratch_types=[
        pltpu.VMEM((SUB, D), jnp.float32),
        pltpu.VMEM((SUB, D), jnp.float32),
        pltpu.VMEM((TAIL, D), jnp.float32),
        pltpu.VMEM((SUB,), jnp.int32),
        pltpu.VMEM((SUB,), jnp.int32),
        pltpu.VMEM((TAIL,), jnp.int32),
        pltpu.VMEM((SEG_PER_TILE, D), jnp.float32),
        pltpu.VMEM((SEG_PER_TILE, D), jnp.float32),
        pltpu.VMEM((SEG_PER_SEARCH,), jnp.float32),
        pltpu.VMEM((32,), jnp.int32),
        pltpu.VMEM((32,), jnp.int32),
        pltpu.SemaphoreType.DMA,
        pltpu.SemaphoreType.DMA,
        pltpu.SemaphoreType.DMA,
        pltpu.VMEM_SHARED((NUM_SEG, D), jnp.float32),
    ],
)
def _sc_accumulate(x_hbm, b_hbm, z_hbm, outs_hbm, outc_hbm,
                   bufA, bufB, bufT, idxA, idxB, idxT, zbuf_v, obuf_v,
                   cbuf_v, mid_v, val_v, semA, semB, semG, acc_sh):
    _sc_body(x_hbm, b_hbm, z_hbm, outs_hbm, outc_hbm,
             bufA, bufB, bufT, idxA, idxB, idxT, zbuf_v, obuf_v, cbuf_v,
             mid_v, val_v, semA, semB, semG, acc_sh)


def _fin_body(s_ref, c_ref, o_ref):
    sums = s_ref[0] + s_ref[1]
    o_ref[...] = sums / jnp.maximum(c_ref[...], 1.0)


def kernel(x, batch):
    batch = batch.astype(jnp.int32)
    zeros = jnp.zeros((SEG_PER_TILE, D), jnp.float32)
    psums, cnts = _sc_accumulate(x, batch, zeros)
    psums = psums.reshape(NC, NUM_SEG, D)
    cnts = cnts.reshape(NUM_SEG, 1)
    return pl.pallas_call(
        _fin_body,
        out_shape=jax.ShapeDtypeStruct((NUM_SEG, D), jnp.float32),
    )(psums, cnts)


# speedup vs baseline: 1.2720x; 1.0049x over previous
import functools

import jax
import jax.numpy as jnp
from jax import lax
from jax.experimental import pallas as pl
from jax.experimental.pallas import tpu as pltpu
from jax.experimental.pallas import tpu_sc as plsc

NUM_SEG = 512
N_ROWS = 320000
D = 128
NC = 2
NS = 16
NW = NC * NS
ROWS_PER_W = N_ROWS // NW
SUB = 128
NFULL = ROWS_PER_W // SUB
TAIL = ROWS_PER_W - NFULL * SUB
NPAIR = NFULL // 2
SEG_PER_TILE = NUM_SEG // NS
SEG_PER_SEARCH = NUM_SEG // NW
SEARCH_STEPS = 19


def _sc_body(x_hbm, b_hbm, z_hbm, out_hbm, pout_hbm,
             bufA, bufB, bufT, idxA, idxB, idxT, zbuf_v, obuf_v, buf2_v,
             cbuf_v, mid_v, val_v, semA, semB, semG, semX, acc_sh):
    c = lax.axis_index("c")
    s = lax.axis_index("s")
    wid = s * NC + c
    seg0 = s * SEG_PER_TILE
    row0 = wid * ROWS_PER_W

    def read(j, buf, idx, sem):
        base = row0 + j * SUB
        pltpu.async_copy(x_hbm.at[pl.ds(base, SUB)], buf, sem)
        pltpu.async_copy(b_hbm.at[pl.ds(base, SUB)], idx, sem)

    def wait(buf, idx, sem):
        pltpu.make_async_copy(x_hbm.at[pl.ds(0, SUB)], buf, sem).wait()
        pltpu.make_async_copy(b_hbm.at[pl.ds(0, SUB)], idx, sem).wait()

    def scatter(buf, idx):
        pltpu.sync_copy(buf, acc_sh.at[idx], add=True)

    read(0, bufA, idxA, semA)
    read(1, bufB, idxB, semB)

    pltpu.sync_copy(z_hbm, zbuf_v)
    pltpu.sync_copy(zbuf_v, acc_sh.at[pl.ds(seg0, SEG_PER_TILE)])
    plsc.subcore_barrier()

    cseg0 = (NUM_SEG // NC) * c + SEG_PER_SEARCH * s
    segv = cseg0 + lax.iota(jnp.int32, 16)
    tgt0 = segv
    tgt1 = segv + 1
    zero = jnp.zeros((16,), jnp.int32)
    nfull = zero + N_ROWS

    def mids(st):
        lo0, hi0, lo1, hi1 = st
        mid0 = jnp.minimum(lax.shift_right_logical(lo0 + hi0, 1), N_ROWS - 1)
        mid1 = jnp.minimum(lax.shift_right_logical(lo1 + hi1, 1), N_ROWS - 1)
        return mid0, mid1

    def issue_gather(st):
        mid0, mid1 = mids(st)
        mid_v[pl.ds(0, 16)] = mid0
        mid_v[pl.ds(16, 16)] = mid1
        pltpu.async_copy(b_hbm.at[mid_v], val_v, semG)

    def apply_step(st):
        lo0, hi0, lo1, hi1 = st
        mid0, mid1 = mids(st)
        pltpu.make_async_copy(b_hbm.at[mid_v], val_v, semG).wait()
        v0 = val_v[pl.ds(0, 16)]
        v1 = val_v[pl.ds(16, 16)]
        p0 = v0 < tgt0
        p1 = v1 < tgt1
        lo0n = jnp.where(p0, mid0 + 1, lo0)
        hi0n = jnp.where(p0, hi0, mid0)
        lo1n = jnp.where(p1, mid1 + 1, lo1)
        hi1n = jnp.where(p1, hi1, mid1)
        return (lo0n, hi0n, lo1n, hi1n)

    st0 = (zero, nfull, zero, nfull)
    issue_gather(st0)

    def body(g, st):
        st = apply_step(st)
        issue_gather(st)
        wait(bufA, idxA, semA)
        scatter(bufA, idxA)

        @pl.when(g < NPAIR - 1)
        def _():
            read(2 * g + 2, bufA, idxA, semA)

        wait(bufB, idxB, semB)
        scatter(bufB, idxB)

        @pl.when(g < NPAIR - 1)
        def _():
            read(2 * g + 3, bufB, idxB, semB)

        return st

    st = lax.fori_loop(0, NPAIR, body, st0)
    tbase = row0 + NFULL * SUB
    pltpu.async_copy(x_hbm.at[pl.ds(tbase, TAIL)], bufT, semA)
    pltpu.async_copy(b_hbm.at[pl.ds(tbase, TAIL)], idxT, semA)
    pltpu.make_async_copy(x_hbm.at[pl.ds(0, TAIL)], bufT, semA).wait()
    pltpu.make_async_copy(b_hbm.at[pl.ds(0, TAIL)], idxT, semA).wait()
    scatter(bufT, idxT)

    lo0, _, lo1, _ = apply_step(st)
    rcp = 1.0 / jnp.maximum((lo1 - lo0).astype(jnp.float32), 1.0)

    plsc.subcore_barrier()
    oc = 1 - c
    orow0 = (NUM_SEG // NC) * oc + SEG_PER_SEARCH * s
    pltpu.sync_copy(acc_sh.at[pl.ds(orow0, SEG_PER_SEARCH)], obuf_v)
    pltpu.sync_copy(obuf_v, pout_hbm.at[pl.ds(orow0, SEG_PER_SEARCH)])
    pltpu.semaphore_signal(semX, 1, core_index=oc)
    pl.semaphore_wait(semX, 1)
    pltpu.sync_copy(pout_hbm.at[pl.ds(cseg0, SEG_PER_SEARCH)], buf2_v)
    pltpu.sync_copy(acc_sh.at[pl.ds(cseg0, SEG_PER_SEARCH)], obuf_v)
    for i in range(SEG_PER_SEARCH):
        rv = jnp.zeros((16,), jnp.float32) + rcp[i]
        for j in range(D // 16):
            sl = pl.ds(j * 16, 16)
            obuf_v[i, sl] = (obuf_v[i, sl] + buf2_v[i, sl]) * rv
    pltpu.sync_copy(obuf_v, out_hbm.at[pl.ds(cseg0, SEG_PER_SEARCH)])


@functools.partial(
    pl.kernel,
    out_type=(
        jax.ShapeDtypeStruct((NUM_SEG, D), jnp.float32),
        jax.ShapeDtypeStruct((NUM_SEG, D), jnp.float32),
    ),
    mesh=plsc.VectorSubcoreMesh(core_axis_name="c", subcore_axis_name="s"),
    scratch_types=[
        pltpu.VMEM((SUB, D), jnp.float32),
        pltpu.VMEM((SUB, D), jnp.float32),
        pltpu.VMEM((TAIL, D), jnp.float32),
        pltpu.VMEM((SUB,), jnp.int32),
        pltpu.VMEM((SUB,), jnp.int32),
        pltpu.VMEM((TAIL,), jnp.int32),
        pltpu.VMEM((SEG_PER_TILE, D), jnp.float32),
        pltpu.VMEM((SEG_PER_SEARCH, D), jnp.float32),
        pltpu.VMEM((SEG_PER_SEARCH, D), jnp.float32),
        pltpu.VMEM((SEG_PER_SEARCH,), jnp.float32),
        pltpu.VMEM((32,), jnp.int32),
        pltpu.VMEM((32,), jnp.int32),
        pltpu.SemaphoreType.DMA,
        pltpu.SemaphoreType.DMA,
        pltpu.SemaphoreType.DMA,
        pltpu.SemaphoreType.REGULAR,
        pltpu.VMEM_SHARED((NUM_SEG, D), jnp.float32),
    ],
)
def _sc_accumulate(x_hbm, b_hbm, z_hbm, out_hbm, pout_hbm,
                   bufA, bufB, bufT, idxA, idxB, idxT, zbuf_v, obuf_v,
                   buf2_v, cbuf_v, mid_v, val_v, semA, semB, semG, semX,
                   acc_sh):
    _sc_body(x_hbm, b_hbm, z_hbm, out_hbm, pout_hbm,
             bufA, bufB, bufT, idxA, idxB, idxT, zbuf_v, obuf_v, buf2_v,
             cbuf_v, mid_v, val_v, semA, semB, semG, semX, acc_sh)


def kernel(x, batch):
    batch = batch.astype(jnp.int32)
    zeros = jnp.zeros((SEG_PER_TILE, D), jnp.float32)
    out, _ = _sc_accumulate(x, batch, zeros)
    return out


# speedup vs baseline: 1.2758x; 1.0031x over previous
import functools

import jax
import jax.numpy as jnp
from jax import lax
from jax.experimental import pallas as pl
from jax.experimental.pallas import tpu as pltpu
from jax.experimental.pallas import tpu_sc as plsc

NUM_SEG = 512
N_ROWS = 320000
D = 128
NC = 2
NS = 16
NW = NC * NS
ROWS_PER_W = N_ROWS // NW
SUB = 128
NFULL = ROWS_PER_W // SUB
TAIL = ROWS_PER_W - NFULL * SUB
NPAIR = NFULL // 2
SEG_PER_TILE = NUM_SEG // NS
SEG_PER_SEARCH = NUM_SEG // NW
SEARCH_STEPS = 19


def _sc_body(x_hbm, b_hbm, out_hbm, pout_hbm,
             bufA, bufB, bufT, idxA, idxB, idxT, zbuf_v, obuf_v, buf2_v,
             cbuf_v, mid_v, val_v, semA, semB, semG, semX, acc_sh):
    c = lax.axis_index("c")
    s = lax.axis_index("s")
    wid = s * NC + c
    seg0 = s * SEG_PER_TILE
    row0 = wid * ROWS_PER_W

    def read(j, buf, idx, sem):
        base = row0 + j * SUB
        pltpu.async_copy(x_hbm.at[pl.ds(base, SUB)], buf, sem)
        pltpu.async_copy(b_hbm.at[pl.ds(base, SUB)], idx, sem)

    def wait(buf, idx, sem):
        pltpu.make_async_copy(x_hbm.at[pl.ds(0, SUB)], buf, sem).wait()
        pltpu.make_async_copy(b_hbm.at[pl.ds(0, SUB)], idx, sem).wait()

    def scatter(buf, idx):
        pltpu.sync_copy(buf, acc_sh.at[idx], add=True)

    read(0, bufA, idxA, semA)
    read(1, bufB, idxB, semB)

    zv = jnp.zeros((16,), jnp.float32)
    for i in range(SEG_PER_TILE):
        for j in range(D // 16):
            zbuf_v[i, pl.ds(j * 16, 16)] = zv
    pltpu.sync_copy(zbuf_v, acc_sh.at[pl.ds(seg0, SEG_PER_TILE)])
    plsc.subcore_barrier()

    cseg0 = (NUM_SEG // NC) * c + SEG_PER_SEARCH * s
    segv = cseg0 + lax.iota(jnp.int32, 16)
    tgt0 = segv
    tgt1 = segv + 1
    zero = jnp.zeros((16,), jnp.int32)
    nfull = zero + N_ROWS

    def mids(st):
        lo0, hi0, lo1, hi1 = st
        mid0 = jnp.minimum(lax.shift_right_logical(lo0 + hi0, 1), N_ROWS - 1)
        mid1 = jnp.minimum(lax.shift_right_logical(lo1 + hi1, 1), N_ROWS - 1)
        return mid0, mid1

    def issue_gather(st):
        mid0, mid1 = mids(st)
        mid_v[pl.ds(0, 16)] = mid0
        mid_v[pl.ds(16, 16)] = mid1
        pltpu.async_copy(b_hbm.at[mid_v], val_v, semG)

    def apply_step(st):
        lo0, hi0, lo1, hi1 = st
        mid0, mid1 = mids(st)
        pltpu.make_async_copy(b_hbm.at[mid_v], val_v, semG).wait()
        v0 = val_v[pl.ds(0, 16)]
        v1 = val_v[pl.ds(16, 16)]
        p0 = v0 < tgt0
        p1 = v1 < tgt1
        lo0n = jnp.where(p0, mid0 + 1, lo0)
        hi0n = jnp.where(p0, hi0, mid0)
        lo1n = jnp.where(p1, mid1 + 1, lo1)
        hi1n = jnp.where(p1, hi1, mid1)
        return (lo0n, hi0n, lo1n, hi1n)

    st0 = (zero, nfull, zero, nfull)
    issue_gather(st0)

    def body(g, st):
        st = apply_step(st)
        issue_gather(st)
        wait(bufA, idxA, semA)
        scatter(bufA, idxA)

        @pl.when(g < NPAIR - 1)
        def _():
            read(2 * g + 2, bufA, idxA, semA)

        wait(bufB, idxB, semB)
        scatter(bufB, idxB)

        @pl.when(g < NPAIR - 1)
        def _():
            read(2 * g + 3, bufB, idxB, semB)

        return st

    st = lax.fori_loop(0, NPAIR, body, st0)
    tbase = row0 + NFULL * SUB
    pltpu.async_copy(x_hbm.at[pl.ds(tbase, TAIL)], bufT, semA)
    pltpu.async_copy(b_hbm.at[pl.ds(tbase, TAIL)], idxT, semA)
    pltpu.make_async_copy(x_hbm.at[pl.ds(0, TAIL)], bufT, semA).wait()
    pltpu.make_async_copy(b_hbm.at[pl.ds(0, TAIL)], idxT, semA).wait()
    scatter(bufT, idxT)

    lo0, _, lo1, _ = apply_step(st)
    rcp = 1.0 / jnp.maximum((lo1 - lo0).astype(jnp.float32), 1.0)

    plsc.subcore_barrier()
    oc = 1 - c
    orow0 = (NUM_SEG // NC) * oc + SEG_PER_SEARCH * s
    pltpu.sync_copy(acc_sh.at[pl.ds(orow0, SEG_PER_SEARCH)], obuf_v)
    pltpu.sync_copy(obuf_v, pout_hbm.at[pl.ds(orow0, SEG_PER_SEARCH)])
    pltpu.semaphore_signal(semX, 1, core_index=oc)
    pl.semaphore_wait(semX, 1)
    pltpu.sync_copy(pout_hbm.at[pl.ds(cseg0, SEG_PER_SEARCH)], buf2_v)
    pltpu.sync_copy(acc_sh.at[pl.ds(cseg0, SEG_PER_SEARCH)], obuf_v)
    for i in range(SEG_PER_SEARCH):
        rv = jnp.zeros((16,), jnp.float32) + rcp[i]
        for j in range(D // 16):
            sl = pl.ds(j * 16, 16)
            obuf_v[i, sl] = (obuf_v[i, sl] + buf2_v[i, sl]) * rv
    pltpu.sync_copy(obuf_v, out_hbm.at[pl.ds(cseg0, SEG_PER_SEARCH)])


@functools.partial(
    pl.kernel,
    out_type=(
        jax.ShapeDtypeStruct((NUM_SEG, D), jnp.float32),
        jax.ShapeDtypeStruct((NUM_SEG, D), jnp.float32),
    ),
    mesh=plsc.VectorSubcoreMesh(core_axis_name="c", subcore_axis_name="s"),
    scratch_types=[
        pltpu.VMEM((SUB, D), jnp.float32),
        pltpu.VMEM((SUB, D), jnp.float32),
        pltpu.VMEM((TAIL, D), jnp.float32),
        pltpu.VMEM((SUB,), jnp.int32),
        pltpu.VMEM((SUB,), jnp.int32),
        pltpu.VMEM((TAIL,), jnp.int32),
        pltpu.VMEM((SEG_PER_TILE, D), jnp.float32),
        pltpu.VMEM((SEG_PER_SEARCH, D), jnp.float32),
        pltpu.VMEM((SEG_PER_SEARCH, D), jnp.float32),
        pltpu.VMEM((SEG_PER_SEARCH,), jnp.float32),
        pltpu.VMEM((32,), jnp.int32),
        pltpu.VMEM((32,), jnp.int32),
        pltpu.SemaphoreType.DMA,
        pltpu.SemaphoreType.DMA,
        pltpu.SemaphoreType.DMA,
        pltpu.SemaphoreType.REGULAR,
        pltpu.VMEM_SHARED((NUM_SEG, D), jnp.float32),
    ],
)
def _sc_accumulate(x_hbm, b_hbm, out_hbm, pout_hbm,
                   bufA, bufB, bufT, idxA, idxB, idxT, zbuf_v, obuf_v,
                   buf2_v, cbuf_v, mid_v, val_v, semA, semB, semG, semX,
                   acc_sh):
    _sc_body(x_hbm, b_hbm, out_hbm, pout_hbm,
             bufA, bufB, bufT, idxA, idxB, idxT, zbuf_v, obuf_v, buf2_v,
             cbuf_v, mid_v, val_v, semA, semB, semG, semX, acc_sh)


def kernel(x, batch):
    batch = batch.astype(jnp.int32)
    out, _ = _sc_accumulate(x, batch)
    return out
